# Initial kernel scaffold; baseline (speedup 1.0000x reference)
import jax
import jax.numpy as jnp
from jax.experimental import pallas as pl


def kernel(msgs_0, msgs_1, index, num_nodes, W1, b1, W2, b2, W_L1, Ws1, bs1, Ws2, bs2):
    raise NotImplementedError("write your pallas kernel here")


# speedup vs baseline: 19.4242x; 19.4242x over previous
import functools

import jax
import jax.numpy as jnp
from jax import lax
from jax.experimental import pallas as pl
from jax.experimental.pallas import tpu as pltpu
from jax.experimental.pallas import tpu_sc as plsc

N = 10000
E = 160000
F = 128
FLAT = 3 * F
INV_SQRT_DEG = 1.0 / 50.0 ** 0.5

NS = 16
CHUNK_A = 80
NCHUNK_A = (E // NS) // CHUNK_A
CHUNK_B = 40
NCHUNK_B = (E // (2 * NS)) // CHUNK_B
NPAD = 10240
ROWS_PER_TILE = NPAD // NS
WB = 128
NWB = ROWS_PER_TILE // WB


def _seg_sum_sc_build():
    mesh = plsc.VectorSubcoreMesh(core_axis_name="c", subcore_axis_name="s")

    @functools.partial(
        pl.kernel,
        mesh=mesh,
        out_type=(
            jax.ShapeDtypeStruct((NPAD, 2 * F), jnp.float32),
            jax.ShapeDtypeStruct((2, NPAD, F), jnp.float32),
        ),
        scratch_types=[
            pltpu.VMEM((CHUNK_A,), jnp.int32),
            pltpu.VMEM((CHUNK_A, F), jnp.float32),
            pltpu.VMEM((CHUNK_B,), jnp.int32),
            pltpu.VMEM((CHUNK_B, F), jnp.float32),
            pltpu.VMEM((WB, F), jnp.float32),
            pltpu.VMEM((WB, F), jnp.float32),
            pltpu.VMEM_SHARED((NPAD, F), jnp.float32),
        ],
    )
    def seg_sum(msgs_hbm, idx_hbm, out01_hbm, out2_hbm,
                idx_a, rows_a, idx_b, rows_b, stage_v, zero_v, acc):
        c = lax.axis_index("c")
        s = lax.axis_index("s")
        r0 = s * ROWS_PER_TILE

        zeros16 = jnp.zeros((16,), jnp.float32)

        def zero_stage():
            def zrow(i, carry):
                def zcol(j, inner):
                    zero_v[i, pl.ds(j * 16, 16)] = zeros16
                    return inner
                return lax.fori_loop(0, F // 16, zcol, carry)
            lax.fori_loop(0, WB, zrow, 0)

        def zero_acc():
            for k in range(NWB):
                pltpu.sync_copy(zero_v, acc.at[pl.ds(r0 + k * WB, WB)])

        zero_stage()
        zero_acc()
        plsc.subcore_barrier()

        base_a = s * (E // NS)
        col_a = c * F

        def body_a(i, carry):
            e0 = base_a + i * CHUNK_A
            pltpu.sync_copy(idx_hbm.at[pl.ds(e0, CHUNK_A)], idx_a)
            pltpu.sync_copy(
                msgs_hbm.at[pl.ds(e0, CHUNK_A), pl.ds(col_a, F)], rows_a)
            pltpu.sync_copy(rows_a, acc.at[idx_a], add=True)
            return carry

        lax.fori_loop(0, NCHUNK_A, body_a, 0)
        plsc.subcore_barrier()

        for k in range(NWB):
            pltpu.sync_copy(acc.at[pl.ds(r0 + k * WB, WB)], stage_v)
            pltpu.sync_copy(
                stage_v, out01_hbm.at[pl.ds(r0 + k * WB, WB), pl.ds(col_a, F)])
        zero_acc()
        plsc.subcore_barrier()

        base_b = c * (E // 2) + s * (E // (2 * NS))

        def body_b(i, carry):
            e0 = base_b + i * CHUNK_B
            pltpu.sync_copy(idx_hbm.at[pl.ds(e0, CHUNK_B)], idx_b)
            pltpu.sync_copy(
                msgs_hbm.at[pl.ds(e0, CHUNK_B), pl.ds(2 * F, F)], rows_b)
            pltpu.sync_copy(rows_b, acc.at[idx_b], add=True)
            return carry

        lax.fori_loop(0, NCHUNK_B, body_b, 0)
        plsc.subcore_barrier()

        for k in range(NWB):
            pltpu.sync_copy(acc.at[pl.ds(r0 + k * WB, WB)], stage_v)
            pltpu.sync_copy(stage_v, out2_hbm.at[c, pl.ds(r0 + k * WB, WB)])

    return seg_sum


_seg_sum_sc = _seg_sum_sc_build()

_BLK = 1000


def _mlp_kernel(denA_ref, den2a_ref, den2b_ref, wl1_ref, bs1_ref, ws2_ref,
                bs2_ref, out_ref, sw_ref):
    @pl.when(pl.program_id(0) == 0)
    def _init():
        h = jax.nn.silu(bs1_ref[...])
        alpha = jnp.dot(h, ws2_ref[...].T,
                        preferred_element_type=jnp.float32) + bs2_ref[...]
        sw_ref[...] = wl1_ref[...].T * (alpha * INV_SQRT_DEG)

    sw = sw_ref[...]
    out_ref[:, 0, :] = jnp.dot(denA_ref[:, :F], sw,
                               preferred_element_type=jnp.float32)
    out_ref[:, 1, :] = jnp.dot(denA_ref[:, F:], sw,
                               preferred_element_type=jnp.float32)
    out_ref[:, 2, :] = jnp.dot(den2a_ref[0] + den2b_ref[0], sw,
                               preferred_element_type=jnp.float32)


_mlp = pl.pallas_call(
    _mlp_kernel,
    grid=(N // _BLK,),
    in_specs=[
        pl.BlockSpec((_BLK, 2 * F), lambda i: (i, 0)),
        pl.BlockSpec((1, _BLK, F), lambda i: (0, i, 0)),
        pl.BlockSpec((1, _BLK, F), lambda i: (1, i, 0)),
        pl.BlockSpec((F, F), lambda i: (0, 0)),
        pl.BlockSpec((1, F), lambda i: (0, 0)),
        pl.BlockSpec((F, F), lambda i: (0, 0)),
        pl.BlockSpec((1, F), lambda i: (0, 0)),
    ],
    out_specs=pl.BlockSpec((_BLK, 3, F), lambda i: (i, 0, 0)),
    out_shape=jax.ShapeDtypeStruct((N, 3, F), jnp.float32),
    scratch_shapes=[pltpu.VMEM((F, F), jnp.float32)],
)


def kernel(msgs_0, msgs_1, index, num_nodes, W1, b1, W2, b2, W_L1, Ws1, bs1,
           Ws2, bs2):
    del msgs_0, num_nodes, W1, b1, W2, b2, Ws1
    denA, den2 = _seg_sum_sc(msgs_1.reshape(E, FLAT), index)
    delta_h1 = _mlp(denA, den2, den2, W_L1, bs1.reshape(1, F), Ws2,
                    bs2.reshape(1, F))
    delta_h0 = jnp.zeros((N, F), dtype=jnp.float32)
    return (delta_h0, delta_h1)


# speedup vs baseline: 35.4733x; 1.8262x over previous
import functools

import jax
import jax.numpy as jnp
from jax import lax
from jax.experimental import pallas as pl
from jax.experimental.pallas import tpu as pltpu
from jax.experimental.pallas import tpu_sc as plsc

N = 10000
E = 160000
F = 128
FLAT = 3 * F
INV_SQRT_DEG = 1.0 / 50.0 ** 0.5

NS = 16
CHUNK_A = 80
NCHUNK_A = (E // NS) // CHUNK_A
CHUNK_B = 40
NCHUNK_B = (E // (2 * NS)) // CHUNK_B
NPAD = 10240
ROWS_PER_TILE = NPAD // NS
WB = 128
NWB = ROWS_PER_TILE // WB


def _seg_sum_sc_build():
    mesh = plsc.VectorSubcoreMesh(core_axis_name="c", subcore_axis_name="s")

    @functools.partial(
        pl.kernel,
        mesh=mesh,
        out_type=(
            jax.ShapeDtypeStruct((NPAD, 2 * F), jnp.float32),
            jax.ShapeDtypeStruct((2, NPAD, F), jnp.float32),
        ),
        scratch_types=[
            pltpu.VMEM((CHUNK_A,), jnp.int32),
            pltpu.VMEM((CHUNK_A, F), jnp.float32),
            pltpu.VMEM((CHUNK_B,), jnp.int32),
            pltpu.VMEM((CHUNK_B, F), jnp.float32),
            pltpu.VMEM((WB, F), jnp.float32),
            pltpu.VMEM((WB, F), jnp.float32),
            pltpu.VMEM_SHARED((NPAD, F), jnp.float32),
        ],
    )
    def seg_sum(msgs_hbm, idx_hbm, out01_hbm, out2_hbm,
                idx_a, rows_a, idx_b, rows_b, stage_v, zero_v, acc):
        c = lax.axis_index("c")
        s = lax.axis_index("s")
        r0 = s * ROWS_PER_TILE

        zeros16 = jnp.zeros((16,), jnp.float32)

        def zero_stage():
            def zrow(i, carry):
                def zcol(j, inner):
                    zero_v[i, pl.ds(j * 16, 16)] = zeros16
                    return inner
                return lax.fori_loop(0, F // 16, zcol, carry)
            lax.fori_loop(0, WB, zrow, 0)

        def zero_acc():
            for k in range(NWB):
                pltpu.sync_copy(zero_v, acc.at[pl.ds(r0 + k * WB, WB)])

        zero_stage()
        zero_acc()
        plsc.subcore_barrier()

        base_a = s * (E // NS)
        col_a = c * F

        def body_a(i, carry):
            e0 = base_a + i * CHUNK_A
            pltpu.sync_copy(idx_hbm.at[pl.ds(e0, CHUNK_A)], idx_a)
            pltpu.sync_copy(msgs_hbm.at[c, pl.ds(e0, CHUNK_A)], rows_a)
            pltpu.sync_copy(rows_a, acc.at[idx_a], add=True)
            return carry

        lax.fori_loop(0, NCHUNK_A, body_a, 0)
        plsc.subcore_barrier()

        for k in range(NWB):
            pltpu.sync_copy(acc.at[pl.ds(r0 + k * WB, WB)], stage_v)
            pltpu.sync_copy(
                stage_v, out01_hbm.at[pl.ds(r0 + k * WB, WB), pl.ds(col_a, F)])
        zero_acc()
        plsc.subcore_barrier()

        base_b = c * (E // 2) + s * (E // (2 * NS))

        def body_b(i, carry):
            e0 = base_b + i * CHUNK_B
            pltpu.sync_copy(idx_hbm.at[pl.ds(e0, CHUNK_B)], idx_b)
            pltpu.sync_copy(msgs_hbm.at[2, pl.ds(e0, CHUNK_B)], rows_b)
            pltpu.sync_copy(rows_b, acc.at[idx_b], add=True)
            return carry

        lax.fori_loop(0, NCHUNK_B, body_b, 0)
        plsc.subcore_barrier()

        for k in range(NWB):
            pltpu.sync_copy(acc.at[pl.ds(r0 + k * WB, WB)], stage_v)
            pltpu.sync_copy(stage_v, out2_hbm.at[c, pl.ds(r0 + k * WB, WB)])

    return seg_sum


_seg_sum_sc = _seg_sum_sc_build()

_BLK = 1000


def _mlp_kernel(denA_ref, den2a_ref, den2b_ref, wl1_ref, bs1_ref, ws2_ref,
                bs2_ref, out_ref, sw_ref):
    @pl.when(pl.program_id(0) == 0)
    def _init():
        h = jax.nn.silu(bs1_ref[...])
        alpha = jnp.dot(h, ws2_ref[...].T,
                        preferred_element_type=jnp.float32) + bs2_ref[...]
        sw_ref[...] = wl1_ref[...].T * (alpha * INV_SQRT_DEG)

    sw = sw_ref[...]
    out_ref[0] = jnp.dot(denA_ref[:, :F], sw,
                         preferred_element_type=jnp.float32)
    out_ref[1] = jnp.dot(denA_ref[:, F:], sw,
                         preferred_element_type=jnp.float32)
    out_ref[2] = jnp.dot(den2a_ref[0] + den2b_ref[0], sw,
                         preferred_element_type=jnp.float32)


_mlp = pl.pallas_call(
    _mlp_kernel,
    grid=(N // _BLK,),
    in_specs=[
        pl.BlockSpec((_BLK, 2 * F), lambda i: (i, 0)),
        pl.BlockSpec((1, _BLK, F), lambda i: (0, i, 0)),
        pl.BlockSpec((1, _BLK, F), lambda i: (1, i, 0)),
        pl.BlockSpec((F, F), lambda i: (0, 0)),
        pl.BlockSpec((1, F), lambda i: (0, 0)),
        pl.BlockSpec((F, F), lambda i: (0, 0)),
        pl.BlockSpec((1, F), lambda i: (0, 0)),
    ],
    out_specs=pl.BlockSpec((3, _BLK, F), lambda i: (0, i, 0)),
    out_shape=jax.ShapeDtypeStruct((3, N, F), jnp.float32),
    scratch_shapes=[pltpu.VMEM((F, F), jnp.float32)],
)


def kernel(msgs_0, msgs_1, index, num_nodes, W1, b1, W2, b2, W_L1, Ws1, bs1,
           Ws2, bs2):
    del msgs_0, num_nodes, W1, b1, W2, b2, Ws1
    msgs_t = jnp.transpose(msgs_1, (1, 0, 2))
    denA, den2 = _seg_sum_sc(msgs_t, index)
    out = _mlp(denA, den2, den2, W_L1, bs1.reshape(1, F), Ws2,
               bs2.reshape(1, F))
    delta_h1 = jnp.transpose(out, (1, 0, 2))
    delta_h0 = jnp.zeros((N, F), dtype=jnp.float32)
    return (delta_h0, delta_h1)


# speedup vs baseline: 74.4934x; 2.1000x over previous
import functools

import jax
import jax.numpy as jnp
from jax import lax
from jax.experimental import pallas as pl
from jax.experimental.pallas import tpu as pltpu
from jax.experimental.pallas import tpu_sc as plsc

N = 10000
E = 160000
F = 128
FLAT = 3 * F
INV_SQRT_DEG = 1.0 / 50.0 ** 0.5

NS = 16
CHUNK_A = 80
NCHUNK_A = (E // NS) // CHUNK_A
CHUNK_B = 40
NCHUNK_B = (E // (2 * NS)) // CHUNK_B
NPAD = 10240
ROWS_PER_TILE = NPAD // NS
WB = 64
NWB = ROWS_PER_TILE // WB


def _seg_sum_sc_build():
    mesh = plsc.VectorSubcoreMesh(core_axis_name="c", subcore_axis_name="s")

    @functools.partial(
        pl.kernel,
        mesh=mesh,
        out_type=(
            jax.ShapeDtypeStruct((NPAD, 2 * F), jnp.float32),
            jax.ShapeDtypeStruct((2, NPAD, F), jnp.float32),
        ),
        scratch_types=[
            pltpu.VMEM((CHUNK_A,), jnp.int32),
            pltpu.VMEM((CHUNK_A,), jnp.int32),
            pltpu.VMEM((CHUNK_A, F), jnp.float32),
            pltpu.VMEM((CHUNK_A, F), jnp.float32),
            pltpu.VMEM((CHUNK_B,), jnp.int32),
            pltpu.VMEM((CHUNK_B,), jnp.int32),
            pltpu.VMEM((CHUNK_B, F), jnp.float32),
            pltpu.VMEM((CHUNK_B, F), jnp.float32),
            pltpu.VMEM((WB, F), jnp.float32),
            pltpu.VMEM((WB, F), jnp.float32),
            pltpu.VMEM_SHARED((NPAD, F), jnp.float32),
            pltpu.SemaphoreType.DMA,
            pltpu.SemaphoreType.DMA,
            pltpu.SemaphoreType.DMA,
            pltpu.SemaphoreType.DMA,
        ],
    )
    def seg_sum(msgs_hbm, idx_hbm, out01_hbm, out2_hbm,
                idx_a0, idx_a1, rows_a0, rows_a1,
                idx_b0, idx_b1, rows_b0, rows_b1,
                stage_v, zero_v, acc, sem_i0, sem_i1, sem_r0, sem_r1):
        c = lax.axis_index("c")
        s = lax.axis_index("s")
        r0 = s * ROWS_PER_TILE

        zeros16 = jnp.zeros((16,), jnp.float32)

        def zero_stage():
            def zrow(i, carry):
                def zcol(j, inner):
                    zero_v[i, pl.ds(j * 16, 16)] = zeros16
                    return inner
                return lax.fori_loop(0, F // 16, zcol, carry)
            lax.fori_loop(0, WB, zrow, 0)

        def zero_acc():
            for k in range(NWB):
                pltpu.sync_copy(zero_v, acc.at[pl.ds(r0 + k * WB, WB)])

        idx_as = (idx_a0, idx_a1)
        rows_as = (rows_a0, rows_a1)
        idx_bs = (idx_b0, idx_b1)
        rows_bs = (rows_b0, rows_b1)
        sem_is = (sem_i0, sem_i1)
        sem_rs = (sem_r0, sem_r1)

        def pipeline(nchunk, chunk, gather_src, idx_bufs, row_bufs):
            def gather(i, b):
                e0 = i * chunk
                pltpu.async_copy(
                    idx_hbm.at[pl.ds(gather_src[0] + e0, chunk)],
                    idx_bufs[b], sem_is[b])
                pltpu.async_copy(
                    msgs_hbm.at[gather_src[1], pl.ds(gather_src[0] + e0, chunk)],
                    row_bufs[b], sem_rs[b])

            def wait(i, b):
                e0 = i * chunk
                pltpu.make_async_copy(
                    idx_hbm.at[pl.ds(gather_src[0] + e0, chunk)],
                    idx_bufs[b], sem_is[b]).wait()
                pltpu.make_async_copy(
                    msgs_hbm.at[gather_src[1], pl.ds(gather_src[0] + e0, chunk)],
                    row_bufs[b], sem_rs[b]).wait()

            gather(0, 0)
            gather(1, 1)

            def body(g, carry):
                i0 = g * 2
                for b in range(2):
                    i = i0 + b
                    wait(i, b)
                    pltpu.sync_copy(row_bufs[b], acc.at[idx_bufs[b]],
                                    add=True)

                    @pl.when(i + 2 < nchunk)
                    def _():
                        gather(i + 2, b)
                return carry

            lax.fori_loop(0, nchunk // 2, body, 0)
            wait(nchunk - 1, 0)
            pltpu.sync_copy(row_bufs[0], acc.at[idx_bufs[0]], add=True)

        zero_stage()
        zero_acc()
        plsc.subcore_barrier()

        base_a = s * (E // NS)
        col_a = c * F
        pipeline(NCHUNK_A, CHUNK_A, (base_a, c), idx_as, rows_as)
        plsc.subcore_barrier()

        for k in range(NWB):
            pltpu.sync_copy(acc.at[pl.ds(r0 + k * WB, WB)], stage_v)
            pltpu.sync_copy(
                stage_v, out01_hbm.at[pl.ds(r0 + k * WB, WB), pl.ds(col_a, F)])
        zero_acc()
        plsc.subcore_barrier()

        base_b = c * (E // 2) + s * (E // (2 * NS))
        pipeline(NCHUNK_B, CHUNK_B, (base_b, 2), idx_bs, rows_bs)
        plsc.subcore_barrier()

        for k in range(NWB):
            pltpu.sync_copy(acc.at[pl.ds(r0 + k * WB, WB)], stage_v)
            pltpu.sync_copy(stage_v, out2_hbm.at[c, pl.ds(r0 + k * WB, WB)])

    return seg_sum


_seg_sum_sc = _seg_sum_sc_build()

_BLK = 1000


def _mlp_kernel(denA_ref, den2a_ref, den2b_ref, wl1_ref, bs1_ref, ws2_ref,
                bs2_ref, out_ref, sw_ref):
    @pl.when(pl.program_id(0) == 0)
    def _init():
        h = jax.nn.silu(bs1_ref[...])
        alpha = jnp.dot(h, ws2_ref[...].T,
                        preferred_element_type=jnp.float32) + bs2_ref[...]
        sw_ref[...] = wl1_ref[...].T * (alpha * INV_SQRT_DEG)

    sw = sw_ref[...]
    out_ref[0] = jnp.dot(denA_ref[:, :F], sw,
                         preferred_element_type=jnp.float32)
    out_ref[1] = jnp.dot(denA_ref[:, F:], sw,
                         preferred_element_type=jnp.float32)
    out_ref[2] = jnp.dot(den2a_ref[0] + den2b_ref[0], sw,
                         preferred_element_type=jnp.float32)


_mlp = pl.pallas_call(
    _mlp_kernel,
    grid=(N // _BLK,),
    in_specs=[
        pl.BlockSpec((_BLK, 2 * F), lambda i: (i, 0)),
        pl.BlockSpec((1, _BLK, F), lambda i: (0, i, 0)),
        pl.BlockSpec((1, _BLK, F), lambda i: (1, i, 0)),
        pl.BlockSpec((F, F), lambda i: (0, 0)),
        pl.BlockSpec((1, F), lambda i: (0, 0)),
        pl.BlockSpec((F, F), lambda i: (0, 0)),
        pl.BlockSpec((1, F), lambda i: (0, 0)),
    ],
    out_specs=pl.BlockSpec((3, _BLK, F), lambda i: (0, i, 0)),
    out_shape=jax.ShapeDtypeStruct((3, N, F), jnp.float32),
    scratch_shapes=[pltpu.VMEM((F, F), jnp.float32)],
)


def kernel(msgs_0, msgs_1, index, num_nodes, W1, b1, W2, b2, W_L1, Ws1, bs1,
           Ws2, bs2):
    del msgs_0, num_nodes, W1, b1, W2, b2, Ws1
    msgs_t = jnp.transpose(msgs_1, (1, 0, 2))
    denA, den2 = _seg_sum_sc(msgs_t, index)
    out = _mlp(denA, den2, den2, W_L1, bs1.reshape(1, F), Ws2,
               bs2.reshape(1, F))
    delta_h1 = jnp.transpose(out, (1, 0, 2))
    delta_h0 = jnp.zeros((N, F), dtype=jnp.float32)
    return (delta_h0, delta_h1)


# speedup vs baseline: 89.9371x; 1.2073x over previous
import functools

import jax
import jax.numpy as jnp
from jax import lax
from jax.experimental import pallas as pl
from jax.experimental.pallas import tpu as pltpu
from jax.experimental.pallas import tpu_sc as plsc

N = 10000
E = 160000
F = 128
FLAT = 3 * F
INV_SQRT_DEG = 1.0 / 50.0 ** 0.5

NS = 16
CH = 128
STRIDE_A = 79 * CH
NA_LAST = (E - (NS - 1) * STRIDE_A) // CH
STRIDE_B = 39 * CH
NB_LAST = (E // 2 - (NS - 1) * STRIDE_B) // CH
NPAD = 10240
ROWS_PER_TILE = NPAD // NS
WB = 64
NWB = ROWS_PER_TILE // WB


def _seg_sum_sc_build():
    mesh = plsc.VectorSubcoreMesh(core_axis_name="c", subcore_axis_name="s")

    @functools.partial(
        pl.kernel,
        mesh=mesh,
        out_type=(
            jax.ShapeDtypeStruct((NPAD, 2 * F), jnp.float32),
            jax.ShapeDtypeStruct((2, NPAD, F), jnp.float32),
        ),
        scratch_types=[
            pltpu.VMEM((CH,), jnp.int32),
            pltpu.VMEM((CH,), jnp.int32),
            pltpu.VMEM((CH, F), jnp.float32),
            pltpu.VMEM((CH, F), jnp.float32),
            pltpu.VMEM((WB, F), jnp.float32),
            pltpu.VMEM_SHARED((NPAD, F), jnp.float32),
            pltpu.SemaphoreType.DMA,
            pltpu.SemaphoreType.DMA,
            pltpu.SemaphoreType.DMA,
            pltpu.SemaphoreType.DMA,
        ],
    )
    def seg_sum(msgs_hbm, idx_hbm, out01_hbm, out2_hbm,
                idx_0, idx_1, rows_0, rows_1, zero_v, acc,
                sem_i0, sem_i1, sem_r0, sem_r1):
        c = lax.axis_index("c")
        s = lax.axis_index("s")
        r0 = s * ROWS_PER_TILE

        zeros16 = jnp.zeros((16,), jnp.float32)

        def zrow(i, carry):
            def zcol(j, inner):
                zero_v[i, pl.ds(j * 16, 16)] = zeros16
                return inner
            return lax.fori_loop(0, F // 16, zcol, carry)

        lax.fori_loop(0, WB, zrow, 0)

        def zero_acc():
            for k in range(NWB):
                pltpu.sync_copy(zero_v, acc.at[pl.ds(r0 + k * WB, WB)])

        idx_bufs = (idx_0, idx_1)
        row_bufs = (rows_0, rows_1)
        sem_is = (sem_i0, sem_i1)
        sem_rs = (sem_r0, sem_r1)

        def pipeline(nchunk, base, plane):
            def gather(i, b):
                e0 = base + i * CH
                pltpu.async_copy(
                    idx_hbm.at[pl.ds(e0, CH)], idx_bufs[b], sem_is[b])
                pltpu.async_copy(
                    msgs_hbm.at[plane, pl.ds(e0, CH)], row_bufs[b], sem_rs[b])

            def wait(i, b):
                e0 = base + i * CH
                pltpu.make_async_copy(
                    idx_hbm.at[pl.ds(e0, CH)], idx_bufs[b], sem_is[b]).wait()
                pltpu.make_async_copy(
                    msgs_hbm.at[plane, pl.ds(e0, CH)], row_bufs[b],
                    sem_rs[b]).wait()

            gather(0, 0)
            gather(1, 1)

            def body(g, carry):
                i0 = g * 2
                for b in range(2):
                    i = i0 + b
                    wait(i, b)
                    pltpu.sync_copy(row_bufs[b], acc.at[idx_bufs[b]],
                                    add=True)

                    @pl.when(i + 2 < nchunk)
                    def _():
                        gather(i + 2, b)
                return carry

            lax.fori_loop(0, nchunk // 2, body, 0)

            @pl.when(nchunk % 2 == 1)
            def _tail():
                wait(nchunk - 1, 0)
                pltpu.sync_copy(row_bufs[0], acc.at[idx_bufs[0]], add=True)

        zero_acc()
        plsc.subcore_barrier()
        n_a = jnp.where(s == NS - 1, NA_LAST, STRIDE_A // CH)
        pipeline(n_a, s * STRIDE_A, c)
        plsc.subcore_barrier()

        pltpu.sync_copy(
            acc.at[pl.ds(r0, ROWS_PER_TILE)],
            out01_hbm.at[pl.ds(r0, ROWS_PER_TILE), pl.ds(c * F, F)])
        zero_acc()
        plsc.subcore_barrier()

        n_b = jnp.where(s == NS - 1, NB_LAST, STRIDE_B // CH)
        pipeline(n_b, c * (E // 2) + s * STRIDE_B, 2)
        plsc.subcore_barrier()

        pltpu.sync_copy(acc.at[pl.ds(r0, ROWS_PER_TILE)],
                        out2_hbm.at[c, pl.ds(r0, ROWS_PER_TILE)])

    return seg_sum


_seg_sum_sc = _seg_sum_sc_build()

_BLK = 1000


def _mlp_kernel(denA_ref, den2a_ref, den2b_ref, wl1_ref, bs1_ref, ws2_ref,
                bs2_ref, out_ref, sw_ref):
    @pl.when(pl.program_id(0) == 0)
    def _init():
        h = jax.nn.silu(bs1_ref[...])
        alpha = jnp.dot(h, ws2_ref[...].T,
                        preferred_element_type=jnp.float32) + bs2_ref[...]
        sw_ref[...] = wl1_ref[...].T * (alpha * INV_SQRT_DEG)

    sw = sw_ref[...]
    out_ref[0] = jnp.dot(denA_ref[:, :F], sw,
                         preferred_element_type=jnp.float32)
    out_ref[1] = jnp.dot(denA_ref[:, F:], sw,
                         preferred_element_type=jnp.float32)
    out_ref[2] = jnp.dot(den2a_ref[0] + den2b_ref[0], sw,
                         preferred_element_type=jnp.float32)


_mlp = pl.pallas_call(
    _mlp_kernel,
    grid=(N // _BLK,),
    in_specs=[
        pl.BlockSpec((_BLK, 2 * F), lambda i: (i, 0)),
        pl.BlockSpec((1, _BLK, F), lambda i: (0, i, 0)),
        pl.BlockSpec((1, _BLK, F), lambda i: (1, i, 0)),
        pl.BlockSpec((F, F), lambda i: (0, 0)),
        pl.BlockSpec((1, F), lambda i: (0, 0)),
        pl.BlockSpec((F, F), lambda i: (0, 0)),
        pl.BlockSpec((1, F), lambda i: (0, 0)),
    ],
    out_specs=pl.BlockSpec((3, _BLK, F), lambda i: (0, i, 0)),
    out_shape=jax.ShapeDtypeStruct((3, N, F), jnp.float32),
    scratch_shapes=[pltpu.VMEM((F, F), jnp.float32)],
)


def kernel(msgs_0, msgs_1, index, num_nodes, W1, b1, W2, b2, W_L1, Ws1, bs1,
           Ws2, bs2):
    del msgs_0, num_nodes, W1, b1, W2, b2, Ws1
    msgs_t = jnp.transpose(msgs_1, (1, 0, 2))
    denA, den2 = _seg_sum_sc(msgs_t, index)
    out = _mlp(denA, den2, den2, W_L1, bs1.reshape(1, F), Ws2,
               bs2.reshape(1, F))
    delta_h1 = jnp.transpose(out, (1, 0, 2))
    delta_h0 = jnp.zeros((N, F), dtype=jnp.float32)
    return (delta_h0, delta_h1)


# speedup vs baseline: 90.2227x; 1.0032x over previous
import functools

import jax
import jax.numpy as jnp
from jax import lax
from jax.experimental import pallas as pl
from jax.experimental.pallas import tpu as pltpu
from jax.experimental.pallas import tpu_sc as plsc

N = 10000
E = 160000
F = 128
FLAT = 3 * F
INV_SQRT_DEG = 1.0 / 50.0 ** 0.5

NS = 16
CH = 80
NBUF = 4
NA = (E // NS) // CH
NB = (E // 2 // NS) // CH
NB_EXTRA = (E // 2 - NS * NB * CH) // CH
NPAD = 10240
ROWS_PER_TILE = NPAD // NS
WB = 32
NWB = ROWS_PER_TILE // WB


def _seg_sum_sc_build():
    mesh = plsc.VectorSubcoreMesh(core_axis_name="c", subcore_axis_name="s")

    @functools.partial(
        pl.kernel,
        mesh=mesh,
        out_type=(
            jax.ShapeDtypeStruct((NPAD, 2 * F), jnp.float32),
            jax.ShapeDtypeStruct((2, NPAD, F), jnp.float32),
        ),
        scratch_types=(
            [pltpu.VMEM((CH,), jnp.int32) for _ in range(NBUF)]
            + [pltpu.VMEM((CH, F), jnp.float32) for _ in range(NBUF)]
            + [pltpu.VMEM((WB, F), jnp.float32),
               pltpu.VMEM_SHARED((NPAD, F), jnp.float32)]
            + [pltpu.SemaphoreType.DMA for _ in range(3 * NBUF)]
        ),
    )
    def seg_sum(msgs_hbm, idx_hbm, out01_hbm, out2_hbm, *scr):
        idx_bufs = scr[0:NBUF]
        row_bufs = scr[NBUF:2 * NBUF]
        zero_v = scr[2 * NBUF]
        acc = scr[2 * NBUF + 1]
        sem_gi = scr[2 * NBUF + 2:2 * NBUF + 2 + NBUF]
        sem_gr = scr[2 * NBUF + 2 + NBUF:2 * NBUF + 2 + 2 * NBUF]
        sem_s = scr[2 * NBUF + 2 + 2 * NBUF:]

        c = lax.axis_index("c")
        s = lax.axis_index("s")
        r0 = s * ROWS_PER_TILE

        zeros16 = jnp.zeros((16,), jnp.float32)

        def zrow(i, carry):
            def zcol(j, inner):
                zero_v[i, pl.ds(j * 16, 16)] = zeros16
                return inner
            return lax.fori_loop(0, F // 16, zcol, carry)

        lax.fori_loop(0, WB, zrow, 0)

        def zero_acc():
            for k in range(NWB):
                pltpu.sync_copy(zero_v, acc.at[pl.ds(r0 + k * WB, WB)])

        def make_ops(base, plane):
            def gather(i, b):
                e0 = base + i * CH
                pltpu.async_copy(
                    idx_hbm.at[pl.ds(e0, CH)], idx_bufs[b], sem_gi[b])
                pltpu.async_copy(
                    msgs_hbm.at[plane, pl.ds(e0, CH)], row_bufs[b], sem_gr[b])

            def wait_gather(i, b):
                e0 = base + i * CH
                pltpu.make_async_copy(
                    idx_hbm.at[pl.ds(e0, CH)], idx_bufs[b], sem_gi[b]).wait()
                pltpu.make_async_copy(
                    msgs_hbm.at[plane, pl.ds(e0, CH)], row_bufs[b],
                    sem_gr[b]).wait()

            def scatter(b):
                pltpu.async_copy(row_bufs[b], acc.at[idx_bufs[b]], sem_s[b],
                                 add=True)

            def wait_scatter(b):
                pltpu.make_async_copy(row_bufs[b], acc.at[idx_bufs[b]],
                                      sem_s[b]).wait()

            return gather, wait_gather, scatter, wait_scatter

        def pipeline(nchunk, base, plane):
            gather, wait_gather, scatter, wait_scatter = make_ops(base, plane)
            gather(0, 0)
            gather(1, 1)

            def body(g, carry):
                for b in range(NBUF):
                    k = g * NBUF + b
                    b2 = (b + 2) % NBUF
                    if b < 2:
                        @pl.when(g > 0)
                        def _():
                            wait_scatter(b2)
                            gather(k + 2, b2)

                        @pl.when(g == 0)
                        def _():
                            gather(k + 2, b2)
                    else:
                        wait_scatter(b2)
                        if b == 3:
                            @pl.when(k + 2 < nchunk)
                            def _():
                                gather(k + 2, b2)
                        else:
                            gather(k + 2, b2)
                    wait_gather(k, b)
                    scatter(b)
                return carry

            lax.fori_loop(0, nchunk // NBUF, body, 0)
            for k in range(nchunk - nchunk % NBUF, nchunk):
                b = k % NBUF
                wait_scatter((k + 2) % NBUF)
                wait_gather(k, b)
                scatter(b)
            wait_scatter((nchunk - 2) % NBUF)
            wait_scatter((nchunk - 1) % NBUF)

        zero_acc()
        plsc.subcore_barrier()
        pipeline(NA, s * (E // NS), c)
        plsc.subcore_barrier()

        pltpu.sync_copy(
            acc.at[pl.ds(r0, ROWS_PER_TILE)],
            out01_hbm.at[pl.ds(r0, ROWS_PER_TILE), pl.ds(c * F, F)])
        zero_acc()
        plsc.subcore_barrier()

        base_b = c * (E // 2)
        pipeline(NB, base_b + s * (NB * CH), 2)

        @pl.when(s < NB_EXTRA)
        def _extra():
            gather, wait_gather, scatter, wait_scatter = make_ops(
                base_b + NS * NB * CH + s * CH, 2)
            gather(0, 0)
            wait_gather(0, 0)
            scatter(0)
            wait_scatter(0)

        plsc.subcore_barrier()

        pltpu.sync_copy(acc.at[pl.ds(r0, ROWS_PER_TILE)],
                        out2_hbm.at[c, pl.ds(r0, ROWS_PER_TILE)])

    return seg_sum


_seg_sum_sc = _seg_sum_sc_build()

_BLK = 1000


def _mlp_kernel(denA_ref, den2a_ref, den2b_ref, wl1_ref, bs1_ref, ws2_ref,
                bs2_ref, out_ref, sw_ref):
    @pl.when(pl.program_id(0) == 0)
    def _init():
        h = jax.nn.silu(bs1_ref[...])
        alpha = jnp.dot(h, ws2_ref[...].T,
                        preferred_element_type=jnp.float32) + bs2_ref[...]
        sw_ref[...] = wl1_ref[...].T * (alpha * INV_SQRT_DEG)

    sw = sw_ref[...]
    out_ref[0] = jnp.dot(denA_ref[:, :F], sw,
                         preferred_element_type=jnp.float32)
    out_ref[1] = jnp.dot(denA_ref[:, F:], sw,
                         preferred_element_type=jnp.float32)
    out_ref[2] = jnp.dot(den2a_ref[0] + den2b_ref[0], sw,
                         preferred_element_type=jnp.float32)


_mlp = pl.pallas_call(
    _mlp_kernel,
    grid=(N // _BLK,),
    in_specs=[
        pl.BlockSpec((_BLK, 2 * F), lambda i: (i, 0)),
        pl.BlockSpec((1, _BLK, F), lambda i: (0, i, 0)),
        pl.BlockSpec((1, _BLK, F), lambda i: (1, i, 0)),
        pl.BlockSpec((F, F), lambda i: (0, 0)),
        pl.BlockSpec((1, F), lambda i: (0, 0)),
        pl.BlockSpec((F, F), lambda i: (0, 0)),
        pl.BlockSpec((1, F), lambda i: (0, 0)),
    ],
    out_specs=pl.BlockSpec((3, _BLK, F), lambda i: (0, i, 0)),
    out_shape=jax.ShapeDtypeStruct((3, N, F), jnp.float32),
    scratch_shapes=[pltpu.VMEM((F, F), jnp.float32)],
)


def kernel(msgs_0, msgs_1, index, num_nodes, W1, b1, W2, b2, W_L1, Ws1, bs1,
           Ws2, bs2):
    del msgs_0, num_nodes, W1, b1, W2, b2, Ws1
    msgs_t = jnp.transpose(msgs_1, (1, 0, 2))
    denA, den2 = _seg_sum_sc(msgs_t, index)
    out = _mlp(denA, den2, den2, W_L1, bs1.reshape(1, F), Ws2,
               bs2.reshape(1, F))
    delta_h1 = jnp.transpose(out, (1, 0, 2))
    delta_h0 = jnp.zeros((N, F), dtype=jnp.float32)
    return (delta_h0, delta_h1)


# speedup vs baseline: 92.1711x; 1.0216x over previous
import functools

import jax
import jax.numpy as jnp
from jax import lax
from jax.experimental import pallas as pl
from jax.experimental.pallas import tpu as pltpu
from jax.experimental.pallas import tpu_sc as plsc

N = 10000
E = 160000
F = 128
FLAT = 3 * F
INV_SQRT_DEG = 1.0 / 50.0 ** 0.5

NS = 16
CH = 80
NBUF = 4
NA = (E // NS) // CH
NB = (E // 2 // NS) // CH
NB_EXTRA = (E // 2 - NS * NB * CH) // CH
NPAD = 10240
ROWS_PER_TILE = NPAD // NS
WB = 32
NWB = ROWS_PER_TILE // WB


def _seg_sum_sc_build():
    mesh = plsc.VectorSubcoreMesh(core_axis_name="c", subcore_axis_name="s")

    @functools.partial(
        pl.kernel,
        mesh=mesh,
        out_type=(
            jax.ShapeDtypeStruct((NPAD, 2 * F), jnp.float32),
            jax.ShapeDtypeStruct((2, NPAD, F), jnp.float32),
        ),
        scratch_types=(
            [pltpu.VMEM((CH,), jnp.int32) for _ in range(NBUF)]
            + [pltpu.VMEM((CH, F), jnp.float32) for _ in range(NBUF)]
            + [pltpu.VMEM((WB, F), jnp.float32),
               pltpu.VMEM_SHARED((NPAD, F), jnp.float32)]
            + [pltpu.SemaphoreType.DMA for _ in range(3 * NBUF)]
        ),
    )
    def seg_sum(msgs_hbm, idx_hbm, out01_hbm, out2_hbm, *scr):
        idx_bufs = scr[0:NBUF]
        row_bufs = scr[NBUF:2 * NBUF]
        zero_v = scr[2 * NBUF]
        acc = scr[2 * NBUF + 1]
        sem_gi = scr[2 * NBUF + 2:2 * NBUF + 2 + NBUF]
        sem_gr = scr[2 * NBUF + 2 + NBUF:2 * NBUF + 2 + 2 * NBUF]
        sem_s = scr[2 * NBUF + 2 + 2 * NBUF:]

        c = lax.axis_index("c")
        s = lax.axis_index("s")
        r0 = s * ROWS_PER_TILE

        zeros16 = jnp.zeros((16,), jnp.float32)

        def zrow(i, carry):
            def zcol(j, inner):
                zero_v[i, pl.ds(j * 16, 16)] = zeros16
                return inner
            return lax.fori_loop(0, F // 16, zcol, carry)

        lax.fori_loop(0, WB, zrow, 0)

        def zero_acc():
            for k in range(NWB):
                pltpu.sync_copy(zero_v, acc.at[pl.ds(r0 + k * WB, WB)])

        def make_ops(base, plane):
            def gather(i, b):
                e0 = base + i * CH
                pltpu.async_copy(
                    idx_hbm.at[pl.ds(e0, CH)], idx_bufs[b], sem_gi[b])
                pltpu.async_copy(
                    msgs_hbm.at[plane, pl.ds(e0, CH)], row_bufs[b], sem_gr[b])

            def wait_gather(i, b):
                e0 = base + i * CH
                pltpu.make_async_copy(
                    idx_hbm.at[pl.ds(e0, CH)], idx_bufs[b], sem_gi[b]).wait()
                pltpu.make_async_copy(
                    msgs_hbm.at[plane, pl.ds(e0, CH)], row_bufs[b],
                    sem_gr[b]).wait()

            def scatter(b):
                pltpu.async_copy(row_bufs[b], acc.at[idx_bufs[b]], sem_s[b],
                                 add=True)

            def wait_scatter(b):
                pltpu.make_async_copy(row_bufs[b], acc.at[idx_bufs[b]],
                                      sem_s[b]).wait()

            return gather, wait_gather, scatter, wait_scatter

        def pipeline(nchunk, base, plane):
            gather, wait_gather, scatter, wait_scatter = make_ops(base, plane)
            gather(0, 0)
            gather(1, 1)

            def body(g, carry):
                for b in range(NBUF):
                    k = g * NBUF + b
                    b2 = (b + 2) % NBUF
                    if b < 2:
                        @pl.when(g > 0)
                        def _():
                            wait_scatter(b2)
                            gather(k + 2, b2)

                        @pl.when(g == 0)
                        def _():
                            gather(k + 2, b2)
                    else:
                        wait_scatter(b2)
                        if b == 3:
                            @pl.when(k + 2 < nchunk)
                            def _():
                                gather(k + 2, b2)
                        else:
                            gather(k + 2, b2)
                    wait_gather(k, b)
                    scatter(b)
                return carry

            lax.fori_loop(0, nchunk // NBUF, body, 0)
            for k in range(nchunk - nchunk % NBUF, nchunk):
                b = k % NBUF
                wait_scatter((k + 2) % NBUF)
                wait_gather(k, b)
                scatter(b)
            wait_scatter((nchunk - 2) % NBUF)
            wait_scatter((nchunk - 1) % NBUF)

        zero_acc()
        plsc.subcore_barrier()
        pipeline(NA, s * (E // NS), c)
        plsc.subcore_barrier()

        pltpu.sync_copy(
            acc.at[pl.ds(r0, ROWS_PER_TILE)],
            out01_hbm.at[pl.ds(r0, ROWS_PER_TILE), pl.ds(c * F, F)])
        zero_acc()
        plsc.subcore_barrier()

        base_b = c * (E // 2)
        pipeline(NB, base_b + s * (NB * CH), 2)

        @pl.when(s < NB_EXTRA)
        def _extra():
            gather, wait_gather, scatter, wait_scatter = make_ops(
                base_b + NS * NB * CH + s * CH, 2)
            gather(0, 0)
            wait_gather(0, 0)
            scatter(0)
            wait_scatter(0)

        plsc.subcore_barrier()

        pltpu.sync_copy(acc.at[pl.ds(r0, ROWS_PER_TILE)],
                        out2_hbm.at[c, pl.ds(r0, ROWS_PER_TILE)])

    return seg_sum


_seg_sum_sc = _seg_sum_sc_build()

_BLK = 2000


def _mlp_kernel(denA_ref, den2a_ref, den2b_ref, wl1_ref, bs1_ref, ws2_ref,
                bs2_ref, out_ref, zero_ref, sw_ref):
    @pl.when(pl.program_id(0) == 0)
    def _init():
        h = jax.nn.silu(bs1_ref[...])
        alpha = jnp.dot(h, ws2_ref[...].T,
                        preferred_element_type=jnp.float32) + bs2_ref[...]
        sw_ref[...] = wl1_ref[...].T * (alpha * INV_SQRT_DEG)

    sw = sw_ref[...]
    out_ref[0] = jnp.dot(denA_ref[:, :F], sw,
                         preferred_element_type=jnp.float32)
    out_ref[1] = jnp.dot(denA_ref[:, F:], sw,
                         preferred_element_type=jnp.float32)
    out_ref[2] = jnp.dot(den2a_ref[0] + den2b_ref[0], sw,
                         preferred_element_type=jnp.float32)
    zero_ref[...] = jnp.zeros_like(zero_ref)


_mlp = pl.pallas_call(
    _mlp_kernel,
    grid=(N // _BLK,),
    in_specs=[
        pl.BlockSpec((_BLK, 2 * F), lambda i: (i, 0)),
        pl.BlockSpec((1, _BLK, F), lambda i: (0, i, 0)),
        pl.BlockSpec((1, _BLK, F), lambda i: (1, i, 0)),
        pl.BlockSpec((F, F), lambda i: (0, 0)),
        pl.BlockSpec((1, F), lambda i: (0, 0)),
        pl.BlockSpec((F, F), lambda i: (0, 0)),
        pl.BlockSpec((1, F), lambda i: (0, 0)),
    ],
    out_specs=[
        pl.BlockSpec((3, _BLK, F), lambda i: (0, i, 0)),
        pl.BlockSpec((_BLK, F), lambda i: (i, 0)),
    ],
    out_shape=[
        jax.ShapeDtypeStruct((3, N, F), jnp.float32),
        jax.ShapeDtypeStruct((N, F), jnp.float32),
    ],
    scratch_shapes=[pltpu.VMEM((F, F), jnp.float32)],
)


def kernel(msgs_0, msgs_1, index, num_nodes, W1, b1, W2, b2, W_L1, Ws1, bs1,
           Ws2, bs2):
    del msgs_0, num_nodes, W1, b1, W2, b2, Ws1
    msgs_t = jnp.transpose(msgs_1, (1, 0, 2))
    denA, den2 = _seg_sum_sc(msgs_t, index)
    out, delta_h0 = _mlp(denA, den2, den2, W_L1, bs1.reshape(1, F), Ws2,
                         bs2.reshape(1, F))
    delta_h1 = jnp.transpose(out, (1, 0, 2))
    return (delta_h0, delta_h1)
